# Initial kernel scaffold; baseline (speedup 1.0000x reference)
import jax
import jax.numpy as jnp
from jax.experimental import pallas as pl


def kernel(x, static_edge_weight, W_gsl, b_gsl, W_de1, b_de1, W_de2, b_de2, W_z, b_z, L_z_W, L_z_b, W_r, b_r, L_r_W, L_r_b, W_h, b_h, L_h_W, L_h_b, W_c, b_c, static_edge_index):
    raise NotImplementedError("write your pallas kernel here")


# speedup vs baseline: 6.4915x; 6.4915x over previous
import functools

import jax
import jax.numpy as jnp
from jax import lax
from jax.experimental import pallas as pl
from jax.experimental.pallas import tpu as pltpu

NP = 1024
NREAL = 1000
FW = 16
FH = 64
NWIN = 4
PREC = jax.lax.Precision.HIGHEST


def _dot(a, b, precision=None):
    return jnp.dot(a, b, preferred_element_type=jnp.float32,
                   precision=PREC if precision is None else precision)


def _dot_nt(a, b):
    return lax.dot_general(a, b, (((1,), (1,)), ((), ())),
                           preferred_element_type=jnp.float32, precision=PREC)


def _prep_body(s_un_ref, s_hat_ref):
    S = s_un_ref[...]
    deg = 1.0 + jnp.sum(S, axis=1)
    dinv = jax.lax.rsqrt(deg)
    Sh = S * dinv[:, None] * dinv[None, :]
    ii = jax.lax.broadcasted_iota(jnp.int32, (NP, NP), 0)
    jj = jax.lax.broadcasted_iota(jnp.int32, (NP, NP), 1)
    d2 = jnp.broadcast_to((dinv * dinv)[:, None], (NP, NP))
    s_hat_ref[...] = Sh + jnp.where((ii == jj) & (ii < NREAL), d2, 0.0)


def _step_body(v_ref, sh_ref, wg1_ref, wg2_ref, bg_ref,
               wde1_ref, bde1_ref, wde2_ref, bde2_ref,
               wz_ref, bz_ref, lz1_ref, lz2_ref, blz_ref,
               wr_ref, br_ref, lr1_ref, lr2_ref, blr_ref,
               wh_ref, bh_ref, lh1_ref, lh2_ref, blh_ref,
               h_out_ref, e1_ref, e2_ref, h_ref):
    t = pl.program_id(1)

    @pl.when(t == 0)
    def _():
        h_ref[...] = jnp.zeros((NP, FH), jnp.float32)

    h = h_ref[...]
    vt = v_ref[0, 0]

    X = _dot(vt, wg1_ref[...]) + _dot(h, wg2_ref[...])
    df = _dot(sh_ref[...], X) + bg_ref[...]

    rmask = (jax.lax.broadcasted_iota(jnp.int32, (NP, 1), 0)
             < NREAL).astype(jnp.float32)
    D1 = rmask * jnp.tanh(_dot(df, wde1_ref[...]) + bde1_ref[...])
    D2 = rmask * jnp.tanh(_dot(df, wde2_ref[...]) + bde2_ref[...])

    Bt = jax.nn.relu(jnp.tanh(_dot_nt(D2, D1) - _dot_nt(D1, D2)))

    c1 = t >= 1
    c2 = t >= 2
    Ms = Bt + jnp.where(c1, e1_ref[...], 0.0) + jnp.where(c2, e2_ref[...], 0.0)
    invk = 1.0 / (1.0 + c1.astype(jnp.float32) + c2.astype(jnp.float32))
    M = Ms * invk
    Am = jnp.where(M > 1e-08, M, 0.0)

    deg = 1.0 + jnp.sum(Am, axis=1)
    dinv = jax.lax.rsqrt(deg)
    u = dinv[:, None] * vt
    aggd = dinv[:, None] * _dot(Am, u) + (dinv * dinv)[:, None] * vt

    Gz = _dot(aggd, wz_ref[...]) + bz_ref[...]
    Gr = _dot(aggd, wr_ref[...]) + br_ref[...]
    Gh = _dot(aggd, wh_ref[...]) + bh_ref[...]
    Z = jax.nn.sigmoid(_dot(Gz, lz1_ref[...]) + _dot(h, lz2_ref[...])
                       + blz_ref[...])
    R = jax.nn.sigmoid(_dot(Gr, lr1_ref[...]) + _dot(h, lr2_ref[...])
                       + blr_ref[...])
    Ht = jnp.tanh(_dot(Gh, lh1_ref[...]) + _dot(h * R, lh2_ref[...])
                  + blh_ref[...])
    hn = Z * h + (1.0 - Z) * Ht

    h_ref[...] = hn
    h_out_ref[0] = hn

    @pl.when(t < NWIN - 1)
    def _():
        e2_ref[...] = e1_ref[...]
        e1_ref[...] = Bt


def _cls_body(h_ref, w_ref, b_ref, o_ref):
    o_ref[...] = _dot(h_ref[...], w_ref[...]) + b_ref[...]


def _full(shape):
    nd = len(shape)
    return pl.BlockSpec(shape, lambda b, t: (0,) * nd)


def _run_steps(v, s_hat, weights, interpret=False):
    B = v.shape[0]
    in_specs = [pl.BlockSpec((1, 1, NP, FW), lambda b, t: (b, t, 0, 0)),
                _full((NP, NP))]
    ops = [v, s_hat]
    for w in weights:
        in_specs.append(_full(w.shape))
        ops.append(w)
    return pl.pallas_call(
        _step_body,
        grid=(B, NWIN),
        in_specs=in_specs,
        out_specs=pl.BlockSpec((1, NP, FH), lambda b, t: (b, 0, 0)),
        out_shape=jax.ShapeDtypeStruct((B, NP, FH), jnp.float32),
        scratch_shapes=[pltpu.VMEM((NP, NP), jnp.float32),
                        pltpu.VMEM((NP, NP), jnp.float32),
                        pltpu.VMEM((NP, FH), jnp.float32)],
        compiler_params=pltpu.CompilerParams(
            dimension_semantics=("arbitrary", "arbitrary")),
        interpret=interpret,
    )(*ops)


def _build_s_un(src, dst, ew):
    flat = dst.astype(jnp.int32) * NP + src.astype(jnp.int32)
    return jnp.zeros((NP * NP,), jnp.float32).at[flat].add(ew)


def _forward_impl(x, static_edge_weight, W_gsl, b_gsl, W_de1, b_de1, W_de2,
                  b_de2, W_z, b_z, L_z_W, L_z_b, W_r, b_r, L_r_W, L_r_b,
                  W_h, b_h, L_h_W, L_h_b, W_c, b_c, static_edge_index,
                  interpret=False):
    B, N, T = x.shape
    xp = jnp.pad(x, ((0, 0), (0, NP - N), (0, 0)))
    v = xp.reshape(B, NP, NWIN, FW).transpose(0, 2, 1, 3)

    src = static_edge_index[0].astype(jnp.int32)
    dst = static_edge_index[1].astype(jnp.int32)
    s_un = _build_s_un(src, dst, static_edge_weight).reshape(NP, NP)

    s_hat = pl.pallas_call(
        _prep_body,
        in_specs=[pl.BlockSpec((NP, NP), lambda: (0, 0))],
        out_specs=pl.BlockSpec((NP, NP), lambda: (0, 0)),
        out_shape=jax.ShapeDtypeStruct((NP, NP), jnp.float32),
        interpret=interpret,
    )(s_un)

    r1 = lambda a: a.reshape(1, -1)
    weights = (W_gsl[:FW], W_gsl[FW:], r1(b_gsl),
               W_de1, r1(b_de1), W_de2, r1(b_de2),
               W_z, r1(b_z), L_z_W[:FH], L_z_W[FH:], r1(L_z_b),
               W_r, r1(b_r), L_r_W[:FH], L_r_W[FH:], r1(L_r_b),
               W_h, r1(b_h), L_h_W[:FH], L_h_W[FH:], r1(L_h_b))

    h = _run_steps(v, s_hat, weights, interpret=interpret)

    wc = jnp.pad(W_c.reshape(N, FH, -1), ((0, NP - N), (0, 0), (0, 0)))
    wc = wc.reshape(NP * FH, -1)
    out = pl.pallas_call(
        _cls_body,
        in_specs=[pl.BlockSpec((B, NP * FH), lambda: (0, 0)),
                  pl.BlockSpec(wc.shape, lambda: (0, 0)),
                  pl.BlockSpec((1, b_c.shape[0]), lambda: (0, 0))],
        out_specs=pl.BlockSpec((B, b_c.shape[0]), lambda: (0, 0)),
        out_shape=jax.ShapeDtypeStruct((B, b_c.shape[0]), jnp.float32),
        interpret=interpret,
    )(h.reshape(B, NP * FH), wc, r1(b_c))
    return out


def kernel(x, static_edge_weight, W_gsl, b_gsl, W_de1, b_de1, W_de2, b_de2,
           W_z, b_z, L_z_W, L_z_b, W_r, b_r, L_r_W, L_r_b, W_h, b_h, L_h_W,
           L_h_b, W_c, b_c, static_edge_index):
    return _forward_impl(x, static_edge_weight, W_gsl, b_gsl, W_de1, b_de1,
                         W_de2, b_de2, W_z, b_z, L_z_W, L_z_b, W_r, b_r,
                         L_r_W, L_r_b, W_h, b_h, L_h_W, L_h_b, W_c, b_c,
                         static_edge_index)


# speedup vs baseline: 7.1511x; 1.1016x over previous
import functools

import jax
import jax.numpy as jnp
from jax import lax
from jax.experimental import pallas as pl
from jax.experimental.pallas import tpu as pltpu

NP = 1024
NREAL = 1000
FW = 16
FH = 64
NWIN = 4
PREC = jax.lax.Precision.HIGHEST


def _dot(a, b, precision=None):
    return jnp.dot(a, b, preferred_element_type=jnp.float32,
                   precision=PREC if precision is None else precision)


def _dot_nt(a, b):
    return lax.dot_general(a, b, (((1,), (1,)), ((), ())),
                           preferred_element_type=jnp.float32, precision=PREC)


def _prep_body(s_un_ref, s_hat_ref):
    S = s_un_ref[...]
    deg = 1.0 + jnp.sum(S, axis=1)
    dinv = jax.lax.rsqrt(deg)
    Sh = S * dinv[:, None] * dinv[None, :]
    ii = jax.lax.broadcasted_iota(jnp.int32, (NP, NP), 0)
    jj = jax.lax.broadcasted_iota(jnp.int32, (NP, NP), 1)
    d2 = jnp.broadcast_to((dinv * dinv)[:, None], (NP, NP))
    s_hat_ref[...] = Sh + jnp.where((ii == jj) & (ii < NREAL), d2, 0.0)


def _step_body(v_ref, sh_ref, wg1_ref, wg2_ref, bg_ref,
               wde1_ref, bde1_ref, wde2_ref, bde2_ref,
               wz_ref, bz_ref, lz1_ref, lz2_ref, blz_ref,
               wr_ref, br_ref, lr1_ref, lr2_ref, blr_ref,
               wh_ref, bh_ref, lh1_ref, lh2_ref, blh_ref,
               h_out_ref, e1_ref, e2_ref, h_ref):
    t = pl.program_id(1)

    @pl.when(t == 0)
    def _():
        h_ref[...] = jnp.zeros((NP, FH), jnp.float32)

    h = h_ref[...]
    vt = v_ref[0, 0]

    X = _dot(vt, wg1_ref[...]) + _dot(h, wg2_ref[...])
    df = _dot(sh_ref[...], X) + bg_ref[...]

    rmask = (jax.lax.broadcasted_iota(jnp.int32, (NP, 1), 0)
             < NREAL).astype(jnp.float32)
    D1 = rmask * jnp.tanh(_dot(df, wde1_ref[...]) + bde1_ref[...])
    D2 = rmask * jnp.tanh(_dot(df, wde2_ref[...]) + bde2_ref[...])

    P = _dot_nt(D2, D1)
    Bt = jax.nn.relu(jnp.tanh(P - P.T))

    c1 = t >= 1
    c2 = t >= 2
    Ms = Bt + jnp.where(c1, e1_ref[...], 0.0) + jnp.where(c2, e2_ref[...], 0.0)
    invk = 1.0 / (1.0 + c1.astype(jnp.float32) + c2.astype(jnp.float32))
    M = Ms * invk
    Am = jnp.where(M > 1e-08, M, 0.0)

    deg = 1.0 + jnp.sum(Am, axis=1)
    dinv = jax.lax.rsqrt(deg)
    u = dinv[:, None] * vt
    aggd = dinv[:, None] * _dot(Am, u) + (dinv * dinv)[:, None] * vt

    Gz = _dot(aggd, wz_ref[...]) + bz_ref[...]
    Gr = _dot(aggd, wr_ref[...]) + br_ref[...]
    Gh = _dot(aggd, wh_ref[...]) + bh_ref[...]
    Z = jax.nn.sigmoid(_dot(Gz, lz1_ref[...]) + _dot(h, lz2_ref[...])
                       + blz_ref[...])
    R = jax.nn.sigmoid(_dot(Gr, lr1_ref[...]) + _dot(h, lr2_ref[...])
                       + blr_ref[...])
    Ht = jnp.tanh(_dot(Gh, lh1_ref[...]) + _dot(h * R, lh2_ref[...])
                  + blh_ref[...])
    hn = Z * h + (1.0 - Z) * Ht

    h_ref[...] = hn
    h_out_ref[0] = hn

    @pl.when(t < NWIN - 1)
    def _():
        e2_ref[...] = e1_ref[...]
        e1_ref[...] = Bt


def _cls_body(h_ref, w_ref, b_ref, o_ref):
    o_ref[...] = _dot(h_ref[...], w_ref[...]) + b_ref[...]


def _full(shape):
    nd = len(shape)
    return pl.BlockSpec(shape, lambda b, t: (0,) * nd)


def _run_steps(v, s_hat, weights, interpret=False):
    B = v.shape[0]
    in_specs = [pl.BlockSpec((1, 1, NP, FW), lambda b, t: (b, t, 0, 0)),
                _full((NP, NP))]
    ops = [v, s_hat]
    for w in weights:
        in_specs.append(_full(w.shape))
        ops.append(w)
    return pl.pallas_call(
        _step_body,
        grid=(B, NWIN),
        in_specs=in_specs,
        out_specs=pl.BlockSpec((1, NP, FH), lambda b, t: (b, 0, 0)),
        out_shape=jax.ShapeDtypeStruct((B, NP, FH), jnp.float32),
        scratch_shapes=[pltpu.VMEM((NP, NP), jnp.float32),
                        pltpu.VMEM((NP, NP), jnp.float32),
                        pltpu.VMEM((NP, FH), jnp.float32)],
        compiler_params=pltpu.CompilerParams(
            dimension_semantics=("arbitrary", "arbitrary")),
        interpret=interpret,
    )(*ops)


def _build_s_un(src, dst, ew):
    flat = dst.astype(jnp.int32) * NP + src.astype(jnp.int32)
    return jnp.zeros((NP * NP,), jnp.float32).at[flat].add(ew)


def _forward_impl(x, static_edge_weight, W_gsl, b_gsl, W_de1, b_de1, W_de2,
                  b_de2, W_z, b_z, L_z_W, L_z_b, W_r, b_r, L_r_W, L_r_b,
                  W_h, b_h, L_h_W, L_h_b, W_c, b_c, static_edge_index,
                  interpret=False):
    B, N, T = x.shape
    xp = jnp.pad(x, ((0, 0), (0, NP - N), (0, 0)))
    v = xp.reshape(B, NP, NWIN, FW).transpose(0, 2, 1, 3)

    src = static_edge_index[0].astype(jnp.int32)
    dst = static_edge_index[1].astype(jnp.int32)
    s_un = _build_s_un(src, dst, static_edge_weight).reshape(NP, NP)

    s_hat = pl.pallas_call(
        _prep_body,
        in_specs=[pl.BlockSpec((NP, NP), lambda: (0, 0))],
        out_specs=pl.BlockSpec((NP, NP), lambda: (0, 0)),
        out_shape=jax.ShapeDtypeStruct((NP, NP), jnp.float32),
        interpret=interpret,
    )(s_un)

    r1 = lambda a: a.reshape(1, -1)
    weights = (W_gsl[:FW], W_gsl[FW:], r1(b_gsl),
               W_de1, r1(b_de1), W_de2, r1(b_de2),
               W_z, r1(b_z), L_z_W[:FH], L_z_W[FH:], r1(L_z_b),
               W_r, r1(b_r), L_r_W[:FH], L_r_W[FH:], r1(L_r_b),
               W_h, r1(b_h), L_h_W[:FH], L_h_W[FH:], r1(L_h_b))

    h = _run_steps(v, s_hat, weights, interpret=interpret)

    wc = jnp.pad(W_c.reshape(N, FH, -1), ((0, NP - N), (0, 0), (0, 0)))
    wc = wc.reshape(NP * FH, -1)
    out = pl.pallas_call(
        _cls_body,
        in_specs=[pl.BlockSpec((B, NP * FH), lambda: (0, 0)),
                  pl.BlockSpec(wc.shape, lambda: (0, 0)),
                  pl.BlockSpec((1, b_c.shape[0]), lambda: (0, 0))],
        out_specs=pl.BlockSpec((B, b_c.shape[0]), lambda: (0, 0)),
        out_shape=jax.ShapeDtypeStruct((B, b_c.shape[0]), jnp.float32),
        interpret=interpret,
    )(h.reshape(B, NP * FH), wc, r1(b_c))
    return out


def kernel(x, static_edge_weight, W_gsl, b_gsl, W_de1, b_de1, W_de2, b_de2,
           W_z, b_z, L_z_W, L_z_b, W_r, b_r, L_r_W, L_r_b, W_h, b_h, L_h_W,
           L_h_b, W_c, b_c, static_edge_index):
    return _forward_impl(x, static_edge_weight, W_gsl, b_gsl, W_de1, b_de1,
                         W_de2, b_de2, W_z, b_z, L_z_W, L_z_b, W_r, b_r,
                         L_r_W, L_r_b, W_h, b_h, L_h_W, L_h_b, W_c, b_c,
                         static_edge_index)


# speedup vs baseline: 17.2567x; 2.4132x over previous
import functools

import jax
import jax.numpy as jnp
from jax import lax
from jax.experimental import pallas as pl
from jax.experimental.pallas import tpu as pltpu
from jax.experimental.pallas import tpu_sc as plsc

NP = 1024
NREAL = 1000
FW = 16
FH = 64
NWIN = 4
PREC = jax.lax.Precision.HIGHEST


def _dot(a, b, precision=None):
    return jnp.dot(a, b, preferred_element_type=jnp.float32,
                   precision=PREC if precision is None else precision)


def _split_bf16(a):
    hi = a.astype(jnp.bfloat16)
    lo = (a - hi.astype(jnp.float32)).astype(jnp.bfloat16)
    return hi, lo


def _dot_nt3(a, b):
    dn = (((1,), (1,)), ((), ()))
    ah, al = _split_bf16(a)
    bh, bl = _split_bf16(b)
    kw = dict(preferred_element_type=jnp.float32,
              precision=jax.lax.Precision.DEFAULT)
    return (lax.dot_general(ah, bh, dn, **kw)
            + lax.dot_general(ah, bl, dn, **kw)
            + lax.dot_general(al, bh, dn, **kw))


def _dot3(ah, al, b):
    bh, bl = _split_bf16(b)
    kw = dict(preferred_element_type=jnp.float32,
              precision=jax.lax.Precision.DEFAULT)
    return (jnp.dot(ah, bh, **kw) + jnp.dot(ah, bl, **kw)
            + jnp.dot(al, bh, **kw))


def _dot33(a, b):
    ah, al = _split_bf16(a)
    return _dot3(ah, al, b)


def _dot1(a, b):
    return jnp.dot(a.astype(jnp.bfloat16), b.astype(jnp.bfloat16),
                   preferred_element_type=jnp.float32,
                   precision=jax.lax.Precision.DEFAULT)


def _prep_body(s_un_ref, s_hi_ref, s_lo_ref):
    S = s_un_ref[...]
    deg = 1.0 + jnp.sum(S, axis=1)
    dinv = jax.lax.rsqrt(deg)
    Sh = S * dinv[:, None] * dinv[None, :]
    ii = jax.lax.broadcasted_iota(jnp.int32, (NP, NP), 0)
    jj = jax.lax.broadcasted_iota(jnp.int32, (NP, NP), 1)
    d2 = jnp.broadcast_to((dinv * dinv)[:, None], (NP, NP))
    Sh = Sh + jnp.where((ii == jj) & (ii < NREAL), d2, 0.0)
    hi, lo = _split_bf16(Sh)
    s_hi_ref[...] = hi
    s_lo_ref[...] = lo


def _step_body(v_ref, shh_ref, shl_ref, wg1_ref, wg2_ref, bg_ref,
               wde1_ref, bde1_ref, wde2_ref, bde2_ref,
               wz_ref, bz_ref, lz1_ref, lz2_ref, blz_ref,
               wr_ref, br_ref, lr1_ref, lr2_ref, blr_ref,
               wh_ref, bh_ref, lh1_ref, lh2_ref, blh_ref,
               h_out_ref, e1_ref, e2_ref, h_ref):
    t = pl.program_id(1)

    @pl.when(t == 0)
    def _():
        h_ref[...] = jnp.zeros((NP, FH), jnp.float32)

    h = h_ref[...]
    vt = v_ref[0, 0]

    X = _dot33(vt, wg1_ref[...]) + _dot33(h, wg2_ref[...])
    df = _dot3(shh_ref[...], shl_ref[...], X) + bg_ref[...]

    rmask = (jax.lax.broadcasted_iota(jnp.int32, (NP, 1), 0)
             < NREAL).astype(jnp.float32)
    D1 = rmask * jnp.tanh(_dot33(df, wde1_ref[...]) + bde1_ref[...])
    D2 = rmask * jnp.tanh(_dot33(df, wde2_ref[...]) + bde2_ref[...])

    P = _dot_nt3(D2, D1)
    Bt = jax.nn.relu(jnp.tanh(P - P.T))

    c1 = t >= 1
    c2 = t >= 2
    Ms = Bt + jnp.where(c2, e1_ref[...] + e2_ref[...],
                        jnp.where(c1, e1_ref[...], 0.0))
    kf = 1.0 + c1.astype(jnp.float32) + c2.astype(jnp.float32)
    invk = 1.0 / kf
    AmS = jnp.where(Ms > 1e-08 * kf, Ms, 0.0)

    deg = 1.0 + invk * jnp.sum(AmS, axis=1)
    dinv = jax.lax.rsqrt(deg)
    u2 = (dinv * invk)[:, None] * vt
    aggd = dinv[:, None] * _dot1(AmS, u2) + (dinv * dinv)[:, None] * vt

    Gz = _dot33(aggd, wz_ref[...]) + bz_ref[...]
    Gr = _dot33(aggd, wr_ref[...]) + br_ref[...]
    Gh = _dot33(aggd, wh_ref[...]) + bh_ref[...]
    Z = jax.nn.sigmoid(_dot33(Gz, lz1_ref[...]) + _dot33(h, lz2_ref[...])
                       + blz_ref[...])
    R = jax.nn.sigmoid(_dot33(Gr, lr1_ref[...]) + _dot33(h, lr2_ref[...])
                       + blr_ref[...])
    Ht = jnp.tanh(_dot33(Gh, lh1_ref[...]) + _dot33(h * R, lh2_ref[...])
                  + blh_ref[...])
    hn = Z * h + (1.0 - Z) * Ht

    h_ref[...] = hn
    h_out_ref[0] = hn

    par = t % 2

    @pl.when((t < NWIN - 1) & (par == 0))
    def _():
        e1_ref[...] = Bt

    @pl.when((t < NWIN - 1) & (par == 1))
    def _():
        e2_ref[...] = Bt


def _cls_body(h_ref, w_ref, b_ref, o_ref):
    o_ref[...] = _dot33(h_ref[...], w_ref[...]) + b_ref[...]


def _full(shape):
    nd = len(shape)
    return pl.BlockSpec(shape, lambda b, t: (0,) * nd)


def _run_steps(v, s_hi, s_lo, weights, interpret=False):
    B = v.shape[0]
    in_specs = [pl.BlockSpec((1, 1, NP, FW), lambda b, t: (b, t, 0, 0)),
                _full((NP, NP)), _full((NP, NP))]
    ops = [v, s_hi, s_lo]
    for w in weights:
        in_specs.append(_full(w.shape))
        ops.append(w)
    return pl.pallas_call(
        _step_body,
        grid=(B, NWIN),
        in_specs=in_specs,
        out_specs=pl.BlockSpec((1, NP, FH), lambda b, t: (b, 0, 0)),
        out_shape=jax.ShapeDtypeStruct((B, NP, FH), jnp.float32),
        scratch_shapes=[pltpu.VMEM((NP, NP), jnp.float32),
                        pltpu.VMEM((NP, NP), jnp.float32),
                        pltpu.VMEM((NP, FH), jnp.float32)],
        compiler_params=pltpu.CompilerParams(
            dimension_semantics=("arbitrary", "arbitrary")),
        interpret=interpret,
    )(*ops)


def _build_s_un(src, dst, ew):
    flat = dst.astype(jnp.int32) * NP + src.astype(jnp.int32)
    return jnp.zeros((NP * NP,), jnp.float32).at[flat].add(ew)


_EPW = 1000
_EPAD = 1008
_HALF = 512
_ZLEN = 2048


def _sc_scatter_body(src_hbm, dst_hbm, ew_hbm, out_hbm,
                     src_v, dst_v, ew_v, idx_v, zbuf, acc):
    c = lax.axis_index("c")
    s = lax.axis_index("s")
    base = s * _EPW
    pltpu.sync_copy(src_hbm.at[pl.ds(base, _EPW)], src_v.at[pl.ds(0, _EPW)])
    pltpu.sync_copy(dst_hbm.at[pl.ds(base, _EPW)], dst_v.at[pl.ds(0, _EPW)])
    pltpu.sync_copy(ew_hbm.at[pl.ds(base, _EPW)], ew_v.at[pl.ds(0, _EPW)])

    row0 = c * _HALF
    trash = _HALF * NP
    lane = lax.broadcasted_iota(jnp.int32, (16,), 0)

    def idx_body(i, carry):
        d = dst_v[pl.ds(i * 16, 16)]
        sv = src_v[pl.ds(i * 16, 16)]
        dr = d - row0
        ok = (dr >= 0) & (dr < _HALF) & ((i * 16 + lane) < _EPW)
        idx_v[pl.ds(i * 16, 16)] = jnp.where(ok, dr * NP + sv, trash)
        return carry

    lax.fori_loop(0, _EPAD // 16, idx_body, 0)

    def z_body(i, carry):
        zbuf[pl.ds(i * 16, 16)] = jnp.zeros((16,), jnp.float32)
        return carry

    lax.fori_loop(0, _ZLEN // 16, z_body, 0)

    rows_per_sub = _HALF // 16

    def zc_body(i, carry):
        pltpu.sync_copy(zbuf,
                        acc.at[pl.ds(s * rows_per_sub * NP + i * _ZLEN,
                                     _ZLEN)])
        return carry

    lax.fori_loop(0, rows_per_sub * NP // _ZLEN, zc_body, 0)

    @pl.when(s == 15)
    def _():
        pltpu.sync_copy(zbuf.at[pl.ds(0, NP)], acc.at[pl.ds(trash, NP)])

    plsc.subcore_barrier()
    pltpu.sync_copy(ew_v, acc.at[idx_v], add=True)
    plsc.subcore_barrier()
    pltpu.sync_copy(
        acc.at[pl.ds(s * rows_per_sub * NP, rows_per_sub * NP)],
        out_hbm.at[pl.ds((row0 + s * rows_per_sub) * NP, rows_per_sub * NP)])


def _build_s_un_sc(src, dst, ew):
    mesh = plsc.VectorSubcoreMesh(core_axis_name="c", subcore_axis_name="s")
    f = functools.partial(
        pl.kernel,
        out_type=jax.ShapeDtypeStruct((NP * NP,), jnp.float32),
        mesh=mesh,
        scratch_types=[
            pltpu.VMEM((_EPAD,), jnp.int32),
            pltpu.VMEM((_EPAD,), jnp.int32),
            pltpu.VMEM((_EPAD,), jnp.float32),
            pltpu.VMEM((_EPAD,), jnp.int32),
            pltpu.VMEM((_ZLEN,), jnp.float32),
            pltpu.VMEM_SHARED(((_HALF + 1) * NP,), jnp.float32),
        ],
    )(_sc_scatter_body)
    return f(src, dst, ew)


def _forward_impl(x, static_edge_weight, W_gsl, b_gsl, W_de1, b_de1, W_de2,
                  b_de2, W_z, b_z, L_z_W, L_z_b, W_r, b_r, L_r_W, L_r_b,
                  W_h, b_h, L_h_W, L_h_b, W_c, b_c, static_edge_index,
                  interpret=False):
    B, N, T = x.shape
    xp = jnp.pad(x, ((0, 0), (0, NP - N), (0, 0)))
    v = xp.reshape(B, NP, NWIN, FW).transpose(0, 2, 1, 3)

    src = static_edge_index[0].astype(jnp.int32)
    dst = static_edge_index[1].astype(jnp.int32)
    s_un = _build_s_un_sc(src, dst, static_edge_weight).reshape(NP, NP)

    s_hi, s_lo = pl.pallas_call(
        _prep_body,
        in_specs=[pl.BlockSpec((NP, NP), lambda: (0, 0))],
        out_specs=[pl.BlockSpec((NP, NP), lambda: (0, 0)),
                   pl.BlockSpec((NP, NP), lambda: (0, 0))],
        out_shape=[jax.ShapeDtypeStruct((NP, NP), jnp.bfloat16),
                   jax.ShapeDtypeStruct((NP, NP), jnp.bfloat16)],
        interpret=interpret,
    )(s_un)

    r1 = lambda a: a.reshape(1, -1)
    weights = (W_gsl[:FW], W_gsl[FW:], r1(b_gsl),
               W_de1, r1(b_de1), W_de2, r1(b_de2),
               W_z, r1(b_z), L_z_W[:FH], L_z_W[FH:], r1(L_z_b),
               W_r, r1(b_r), L_r_W[:FH], L_r_W[FH:], r1(L_r_b),
               W_h, r1(b_h), L_h_W[:FH], L_h_W[FH:], r1(L_h_b))

    h = _run_steps(v, s_hi, s_lo, weights, interpret=interpret)

    wc = jnp.pad(W_c.reshape(N, FH, -1), ((0, NP - N), (0, 0), (0, 0)))
    wc = wc.reshape(NP * FH, -1)
    out = pl.pallas_call(
        _cls_body,
        in_specs=[pl.BlockSpec((B, NP * FH), lambda: (0, 0)),
                  pl.BlockSpec(wc.shape, lambda: (0, 0)),
                  pl.BlockSpec((1, b_c.shape[0]), lambda: (0, 0))],
        out_specs=pl.BlockSpec((B, b_c.shape[0]), lambda: (0, 0)),
        out_shape=jax.ShapeDtypeStruct((B, b_c.shape[0]), jnp.float32),
        interpret=interpret,
    )(h.reshape(B, NP * FH), wc, r1(b_c))
    return out


def kernel(x, static_edge_weight, W_gsl, b_gsl, W_de1, b_de1, W_de2, b_de2,
           W_z, b_z, L_z_W, L_z_b, W_r, b_r, L_r_W, L_r_b, W_h, b_h, L_h_W,
           L_h_b, W_c, b_c, static_edge_index):
    return _forward_impl(x, static_edge_weight, W_gsl, b_gsl, W_de1, b_de1,
                         W_de2, b_de2, W_z, b_z, L_z_W, L_z_b, W_r, b_r,
                         L_r_W, L_r_b, W_h, b_h, L_h_W, L_h_b, W_c, b_c,
                         static_edge_index)


# speedup vs baseline: 19.8731x; 1.1516x over previous
import functools

import jax
import jax.numpy as jnp
from jax import lax
from jax.experimental import pallas as pl
from jax.experimental.pallas import tpu as pltpu
from jax.experimental.pallas import tpu_sc as plsc

NP = 1024
NREAL = 1000
FW = 16
FH = 64
NWIN = 4
PREC = jax.lax.Precision.HIGHEST


def _dot(a, b, precision=None):
    return jnp.dot(a, b, preferred_element_type=jnp.float32,
                   precision=PREC if precision is None else precision)


def _split_bf16(a):
    hi = a.astype(jnp.bfloat16)
    lo = (a - hi.astype(jnp.float32)).astype(jnp.bfloat16)
    return hi, lo


def _dot_nt3(a, b):
    dn = (((1,), (1,)), ((), ()))
    ah, al = _split_bf16(a)
    bh, bl = _split_bf16(b)
    kw = dict(preferred_element_type=jnp.float32,
              precision=jax.lax.Precision.DEFAULT)
    return (lax.dot_general(ah, bh, dn, **kw)
            + lax.dot_general(ah, bl, dn, **kw)
            + lax.dot_general(al, bh, dn, **kw))


def _dot3(ah, al, b):
    bh, bl = _split_bf16(b)
    kw = dict(preferred_element_type=jnp.float32,
              precision=jax.lax.Precision.DEFAULT)
    return (jnp.dot(ah, bh, **kw) + jnp.dot(ah, bl, **kw)
            + jnp.dot(al, bh, **kw))


def _dot33(a, b):
    ah, al = _split_bf16(a)
    return _dot3(ah, al, b)


def _dot3p(ah, al, bh, bl):
    kw = dict(preferred_element_type=jnp.float32,
              precision=jax.lax.Precision.DEFAULT)
    return (jnp.dot(ah, bh, **kw) + jnp.dot(ah, bl, **kw)
            + jnp.dot(al, bh, **kw))


def _dot1(a, b):
    return jnp.dot(a.astype(jnp.bfloat16), b.astype(jnp.bfloat16),
                   preferred_element_type=jnp.float32,
                   precision=jax.lax.Precision.DEFAULT)


def _prep_body(s_un_ref, s_hi_ref, s_lo_ref):
    S = s_un_ref[...]
    deg = 1.0 + jnp.sum(S, axis=1)
    dinv = jax.lax.rsqrt(deg)
    Sh = S * dinv[:, None] * dinv[None, :]
    ii = jax.lax.broadcasted_iota(jnp.int32, (NP, NP), 0)
    jj = jax.lax.broadcasted_iota(jnp.int32, (NP, NP), 1)
    d2 = jnp.broadcast_to((dinv * dinv)[:, None], (NP, NP))
    Sh = Sh + jnp.where((ii == jj) & (ii < NREAL), d2, 0.0)
    hi, lo = _split_bf16(Sh)
    s_hi_ref[...] = hi
    s_lo_ref[...] = lo


def _step_body(v_ref, shh_ref, shl_ref, wg1_ref, wg2_ref, bg_ref,
               wde1_ref, bde1_ref, wde2_ref, bde2_ref,
               wz_ref, bz_ref, lz1_ref, lz2_ref, blz_ref,
               wr_ref, br_ref, lr1_ref, lr2_ref, blr_ref,
               wh_ref, bh_ref, lh1_ref, lh2_ref, blh_ref,
               h_out_ref, e1_ref, e2_ref, h_ref):
    t = pl.program_id(1)

    @pl.when(t == 0)
    def _():
        h_ref[...] = jnp.zeros((NP, FH), jnp.float32)

    h = h_ref[...]
    vt = v_ref[0, 0]
    hh, hl = _split_bf16(h)

    X = _dot33(vt, wg1_ref[...]) + _dot3(hh, hl, wg2_ref[...])
    df = _dot3(shh_ref[...], shl_ref[...], X) + bg_ref[...]

    rmask = (jax.lax.broadcasted_iota(jnp.int32, (NP, 1), 0)
             < NREAL).astype(jnp.float32)
    dfh, dfl = _split_bf16(df)
    D1 = rmask * jnp.tanh(_dot3(dfh, dfl, wde1_ref[...]) + bde1_ref[...])
    D2 = rmask * jnp.tanh(_dot3(dfh, dfl, wde2_ref[...]) + bde2_ref[...])

    P = _dot_nt3(D2, D1)
    Bt = jax.nn.relu(jnp.tanh(P - P.T))

    c1 = t >= 1
    c2 = t >= 2
    Ms = Bt + jnp.where(c2, e1_ref[...] + e2_ref[...],
                        jnp.where(c1, e1_ref[...], 0.0))
    kf = 1.0 + c1.astype(jnp.float32) + c2.astype(jnp.float32)
    invk = 1.0 / kf
    AmS = jnp.where(Ms > 1e-08 * kf, Ms, 0.0)
    Ab = AmS.astype(jnp.bfloat16)

    deg = 1.0 + invk * jnp.sum(AmS, axis=1)
    dinv = jax.lax.rsqrt(deg)
    u2 = (dinv * invk)[:, None] * vt
    pre = jnp.dot(Ab, u2.astype(jnp.bfloat16),
                  preferred_element_type=jnp.float32,
                  precision=jax.lax.Precision.DEFAULT)
    aggd = dinv[:, None] * pre + (dinv * dinv)[:, None] * vt

    gh_, gl_ = _split_bf16(aggd)
    Gz = _dot3(gh_, gl_, wz_ref[...]) + bz_ref[...]
    Gr = _dot3(gh_, gl_, wr_ref[...]) + br_ref[...]
    Gh = _dot3(gh_, gl_, wh_ref[...]) + bh_ref[...]
    Z = jax.nn.sigmoid(_dot33(Gz, lz1_ref[...]) + _dot3(hh, hl, lz2_ref[...])
                       + blz_ref[...])
    R = jax.nn.sigmoid(_dot33(Gr, lr1_ref[...]) + _dot3(hh, hl, lr2_ref[...])
                       + blr_ref[...])
    Ht = jnp.tanh(_dot33(Gh, lh1_ref[...]) + _dot33(h * R, lh2_ref[...])
                  + blh_ref[...])
    hn = Z * h + (1.0 - Z) * Ht

    h_ref[...] = hn
    h_out_ref[0] = hn

    par = t % 2

    @pl.when((t < NWIN - 1) & (par == 0))
    def _():
        e1_ref[...] = Bt

    @pl.when((t < NWIN - 1) & (par == 1))
    def _():
        e2_ref[...] = Bt


def _cls_body(h_ref, w_ref, b_ref, o_ref):
    hh, hl = _split_bf16(h_ref[...])
    wh, wl = _split_bf16(w_ref[...])
    o_ref[...] = _dot3p(hh, hl, wh, wl) + b_ref[...]


def _full(shape):
    nd = len(shape)
    return pl.BlockSpec(shape, lambda b, t: (0,) * nd)


def _run_steps(v, s_hi, s_lo, weights, interpret=False):
    B = v.shape[0]
    in_specs = [pl.BlockSpec((1, 1, NP, FW), lambda b, t: (b, t, 0, 0)),
                _full((NP, NP)), _full((NP, NP))]
    ops = [v, s_hi, s_lo]
    for w in weights:
        in_specs.append(_full(w.shape))
        ops.append(w)
    return pl.pallas_call(
        _step_body,
        grid=(B, NWIN),
        in_specs=in_specs,
        out_specs=pl.BlockSpec((1, NP, FH), lambda b, t: (b, 0, 0)),
        out_shape=jax.ShapeDtypeStruct((B, NP, FH), jnp.float32),
        scratch_shapes=[pltpu.VMEM((NP, NP), jnp.float32),
                        pltpu.VMEM((NP, NP), jnp.float32),
                        pltpu.VMEM((NP, FH), jnp.float32)],
        compiler_params=pltpu.CompilerParams(
            dimension_semantics=("arbitrary", "arbitrary")),
        interpret=interpret,
    )(*ops)


def _build_s_un(src, dst, ew):
    flat = dst.astype(jnp.int32) * NP + src.astype(jnp.int32)
    return jnp.zeros((NP * NP,), jnp.float32).at[flat].add(ew)


_EPW = 1000
_EPAD = 1008
_HALF = 512
_ZLEN = 32 * NP


def _sc_scatter_body(src_hbm, dst_hbm, ew_hbm, out_hbm,
                     src_v, dst_v, ew_v, idx_v, zbuf, acc, sem):
    c = lax.axis_index("c")
    s = lax.axis_index("s")
    base = s * _EPW
    cp1 = pltpu.async_copy(src_hbm.at[pl.ds(base, _EPW)],
                           src_v.at[pl.ds(0, _EPW)], sem)
    cp2 = pltpu.async_copy(dst_hbm.at[pl.ds(base, _EPW)],
                           dst_v.at[pl.ds(0, _EPW)], sem)
    cp3 = pltpu.async_copy(ew_hbm.at[pl.ds(base, _EPW)],
                           ew_v.at[pl.ds(0, _EPW)], sem)

    def z_body(i, carry):
        zbuf[pl.ds(i * 16, 16)] = jnp.zeros((16,), jnp.float32)
        return carry

    lax.fori_loop(0, _ZLEN // 16, z_body, 0)

    rows_per_sub = _HALF // 16
    pltpu.sync_copy(zbuf, acc.at[pl.ds(s * rows_per_sub * NP, _ZLEN)])

    row0 = c * _HALF
    trash = _HALF * NP
    lane = lax.broadcasted_iota(jnp.int32, (16,), 0)

    @pl.when(s == 15)
    def _():
        pltpu.sync_copy(zbuf.at[pl.ds(0, NP)], acc.at[pl.ds(trash, NP)])

    cp1.wait()
    cp2.wait()
    cp3.wait()

    def idx_body(i, carry):
        d = dst_v[pl.ds(i * 16, 16)]
        sv = src_v[pl.ds(i * 16, 16)]
        dr = d - row0
        ok = (dr >= 0) & (dr < _HALF) & ((i * 16 + lane) < _EPW)
        idx_v[pl.ds(i * 16, 16)] = jnp.where(ok, dr * NP + sv, trash)
        return carry

    lax.fori_loop(0, _EPAD // 16, idx_body, 0)

    plsc.subcore_barrier()
    pltpu.sync_copy(ew_v, acc.at[idx_v], add=True)
    plsc.subcore_barrier()
    pltpu.sync_copy(
        acc.at[pl.ds(s * rows_per_sub * NP, rows_per_sub * NP)],
        out_hbm.at[pl.ds((row0 + s * rows_per_sub) * NP, rows_per_sub * NP)])


def _build_s_un_sc(src, dst, ew):
    mesh = plsc.VectorSubcoreMesh(core_axis_name="c", subcore_axis_name="s")
    f = functools.partial(
        pl.kernel,
        out_type=jax.ShapeDtypeStruct((NP * NP,), jnp.float32),
        mesh=mesh,
        scratch_types=[
            pltpu.VMEM((_EPAD,), jnp.int32),
            pltpu.VMEM((_EPAD,), jnp.int32),
            pltpu.VMEM((_EPAD,), jnp.float32),
            pltpu.VMEM((_EPAD,), jnp.int32),
            pltpu.VMEM((_ZLEN,), jnp.float32),
            pltpu.VMEM_SHARED(((_HALF + 1) * NP,), jnp.float32),
            pltpu.SemaphoreType.DMA,
        ],
    )(_sc_scatter_body)
    return f(src, dst, ew)


def _forward_impl(x, static_edge_weight, W_gsl, b_gsl, W_de1, b_de1, W_de2,
                  b_de2, W_z, b_z, L_z_W, L_z_b, W_r, b_r, L_r_W, L_r_b,
                  W_h, b_h, L_h_W, L_h_b, W_c, b_c, static_edge_index,
                  interpret=False):
    B, N, T = x.shape
    xp = jnp.pad(x, ((0, 0), (0, NP - N), (0, 0)))
    v = xp.reshape(B, NP, NWIN, FW).transpose(0, 2, 1, 3)

    src = static_edge_index[0].astype(jnp.int32)
    dst = static_edge_index[1].astype(jnp.int32)
    s_un = _build_s_un_sc(src, dst, static_edge_weight).reshape(NP, NP)

    s_hi, s_lo = pl.pallas_call(
        _prep_body,
        in_specs=[pl.BlockSpec((NP, NP), lambda: (0, 0))],
        out_specs=[pl.BlockSpec((NP, NP), lambda: (0, 0)),
                   pl.BlockSpec((NP, NP), lambda: (0, 0))],
        out_shape=[jax.ShapeDtypeStruct((NP, NP), jnp.bfloat16),
                   jax.ShapeDtypeStruct((NP, NP), jnp.bfloat16)],
        interpret=interpret,
    )(s_un)

    r1 = lambda a: a.reshape(1, -1)
    weights = (W_gsl[:FW], W_gsl[FW:], r1(b_gsl),
               W_de1, r1(b_de1), W_de2, r1(b_de2),
               W_z, r1(b_z), L_z_W[:FH], L_z_W[FH:], r1(L_z_b),
               W_r, r1(b_r), L_r_W[:FH], L_r_W[FH:], r1(L_r_b),
               W_h, r1(b_h), L_h_W[:FH], L_h_W[FH:], r1(L_h_b))

    h = _run_steps(v, s_hi, s_lo, weights, interpret=interpret)

    hf = h[:, :N, :].reshape(B, N * FH)
    out = pl.pallas_call(
        _cls_body,
        in_specs=[pl.BlockSpec((B, N * FH), lambda: (0, 0)),
                  pl.BlockSpec(W_c.shape, lambda: (0, 0)),
                  pl.BlockSpec((1, b_c.shape[0]), lambda: (0, 0))],
        out_specs=pl.BlockSpec((B, b_c.shape[0]), lambda: (0, 0)),
        out_shape=jax.ShapeDtypeStruct((B, b_c.shape[0]), jnp.float32),
        interpret=interpret,
    )(hf, W_c, r1(b_c))
    return out


def kernel(x, static_edge_weight, W_gsl, b_gsl, W_de1, b_de1, W_de2, b_de2,
           W_z, b_z, L_z_W, L_z_b, W_r, b_r, L_r_W, L_r_b, W_h, b_h, L_h_W,
           L_h_b, W_c, b_c, static_edge_index):
    return _forward_impl(x, static_edge_weight, W_gsl, b_gsl, W_de1, b_de1,
                         W_de2, b_de2, W_z, b_z, L_z_W, L_z_b, W_r, b_r,
                         L_r_W, L_r_b, W_h, b_h, L_h_W, L_h_b, W_c, b_c,
                         static_edge_index)


# speedup vs baseline: 19.9482x; 1.0038x over previous
import functools

import jax
import jax.numpy as jnp
from jax import lax
from jax.experimental import pallas as pl
from jax.experimental.pallas import tpu as pltpu
from jax.experimental.pallas import tpu_sc as plsc

NP = 1024
NREAL = 1000
FW = 16
FH = 64
NWIN = 4
PREC = jax.lax.Precision.HIGHEST


def _dot(a, b, precision=None):
    return jnp.dot(a, b, preferred_element_type=jnp.float32,
                   precision=PREC if precision is None else precision)


def _split_bf16(a):
    hi = a.astype(jnp.bfloat16)
    lo = (a - hi.astype(jnp.float32)).astype(jnp.bfloat16)
    return hi, lo


def _dot_nt3(a, b):
    dn = (((1,), (1,)), ((), ()))
    ah, al = _split_bf16(a)
    bh, bl = _split_bf16(b)
    kw = dict(preferred_element_type=jnp.float32,
              precision=jax.lax.Precision.DEFAULT)
    return (lax.dot_general(ah, bh, dn, **kw)
            + lax.dot_general(ah, bl, dn, **kw)
            + lax.dot_general(al, bh, dn, **kw))


def _dot3(ah, al, b):
    bh, bl = _split_bf16(b)
    kw = dict(preferred_element_type=jnp.float32,
              precision=jax.lax.Precision.DEFAULT)
    return (jnp.dot(ah, bh, **kw) + jnp.dot(ah, bl, **kw)
            + jnp.dot(al, bh, **kw))


def _dot33(a, b):
    ah, al = _split_bf16(a)
    return _dot3(ah, al, b)


def _dot3p(ah, al, bh, bl):
    kw = dict(preferred_element_type=jnp.float32,
              precision=jax.lax.Precision.DEFAULT)
    return (jnp.dot(ah, bh, **kw) + jnp.dot(ah, bl, **kw)
            + jnp.dot(al, bh, **kw))


def _dot1(a, b):
    return jnp.dot(a.astype(jnp.bfloat16), b.astype(jnp.bfloat16),
                   preferred_element_type=jnp.float32,
                   precision=jax.lax.Precision.DEFAULT)


def _prep_body(s_un_ref, s_hi_ref, s_lo_ref):
    S = s_un_ref[...]
    deg = 1.0 + jnp.sum(S, axis=1)
    dinv = jax.lax.rsqrt(deg)
    Sh = S * dinv[:, None] * dinv[None, :]
    ii = jax.lax.broadcasted_iota(jnp.int32, (NP, NP), 0)
    jj = jax.lax.broadcasted_iota(jnp.int32, (NP, NP), 1)
    d2 = jnp.broadcast_to((dinv * dinv)[:, None], (NP, NP))
    Sh = Sh + jnp.where((ii == jj) & (ii < NREAL), d2, 0.0)
    hi, lo = _split_bf16(Sh)
    s_hi_ref[...] = hi
    s_lo_ref[...] = lo


def _step_body(v_ref, shh_ref, shl_ref, wg1_ref, wg2_ref, bg_ref,
               wde1_ref, bde1_ref, wde2_ref, bde2_ref,
               wz_ref, bz_ref, lz1_ref, lz2_ref, blz_ref,
               wr_ref, br_ref, lr1_ref, lr2_ref, blr_ref,
               wh_ref, bh_ref, lh1_ref, lh2_ref, blh_ref,
               h_out_ref, e1_ref, e2_ref, h_ref):
    t = pl.program_id(1)

    @pl.when(t == 0)
    def _():
        h_ref[...] = jnp.zeros((NP, FH), jnp.float32)

    h = h_ref[...]
    vt = v_ref[0, 0]
    hh, hl = _split_bf16(h)

    X = _dot33(vt, wg1_ref[...]) + _dot3(hh, hl, wg2_ref[...])
    df = _dot3(shh_ref[...], shl_ref[...], X) + bg_ref[...]

    rmask = (jax.lax.broadcasted_iota(jnp.int32, (NP, 1), 0)
             < NREAL).astype(jnp.float32)
    dfh, dfl = _split_bf16(df)
    D1 = rmask * jnp.tanh(_dot3(dfh, dfl, wde1_ref[...]) + bde1_ref[...])
    D2 = rmask * jnp.tanh(_dot3(dfh, dfl, wde2_ref[...]) + bde2_ref[...])

    P = _dot_nt3(D2, D1)
    Bt = jax.nn.relu(jnp.tanh(P - P.T))

    c1 = t >= 1
    c2 = t >= 2
    E1 = e1_ref[...].astype(jnp.float32)
    E2 = e2_ref[...].astype(jnp.float32)
    Ms = Bt + jnp.where(c2, E1 + E2, jnp.where(c1, E1, 0.0))
    kf = 1.0 + c1.astype(jnp.float32) + c2.astype(jnp.float32)
    invk = 1.0 / kf
    Ab = Ms.astype(jnp.bfloat16)

    deg = 1.0 + invk * jnp.sum(Ms, axis=1)
    dinv = jax.lax.rsqrt(deg)
    u2 = (dinv * invk)[:, None] * vt
    pre = jnp.dot(Ab, u2.astype(jnp.bfloat16),
                  preferred_element_type=jnp.float32,
                  precision=jax.lax.Precision.DEFAULT)
    aggd = dinv[:, None] * pre + (dinv * dinv)[:, None] * vt

    gh_, gl_ = _split_bf16(aggd)
    Gz = _dot3(gh_, gl_, wz_ref[...]) + bz_ref[...]
    Gr = _dot3(gh_, gl_, wr_ref[...]) + br_ref[...]
    Gh = _dot3(gh_, gl_, wh_ref[...]) + bh_ref[...]
    Z = jax.nn.sigmoid(_dot33(Gz, lz1_ref[...]) + _dot3(hh, hl, lz2_ref[...])
                       + blz_ref[...])
    R = jax.nn.sigmoid(_dot33(Gr, lr1_ref[...]) + _dot3(hh, hl, lr2_ref[...])
                       + blr_ref[...])
    Ht = jnp.tanh(_dot33(Gh, lh1_ref[...]) + _dot33(h * R, lh2_ref[...])
                  + blh_ref[...])
    hn = Z * h + (1.0 - Z) * Ht

    h_ref[...] = hn
    h_out_ref[0] = hn

    par = t % 2

    @pl.when((t < NWIN - 1) & (par == 0))
    def _():
        e1_ref[...] = Bt.astype(jnp.bfloat16)

    @pl.when((t < NWIN - 1) & (par == 1))
    def _():
        e2_ref[...] = Bt.astype(jnp.bfloat16)


def _cls_body(h_ref, w_ref, b_ref, o_ref):
    hh, hl = _split_bf16(h_ref[...])
    wh, wl = _split_bf16(w_ref[...])
    o_ref[...] = _dot3p(hh, hl, wh, wl) + b_ref[...]


def _full(shape):
    nd = len(shape)
    return pl.BlockSpec(shape, lambda b, t: (0,) * nd)


def _run_steps(v, s_hi, s_lo, weights, interpret=False):
    B = v.shape[0]
    in_specs = [pl.BlockSpec((1, 1, NP, FW), lambda b, t: (b, t, 0, 0)),
                _full((NP, NP)), _full((NP, NP))]
    ops = [v, s_hi, s_lo]
    for w in weights:
        in_specs.append(_full(w.shape))
        ops.append(w)
    return pl.pallas_call(
        _step_body,
        grid=(B, NWIN),
        in_specs=in_specs,
        out_specs=pl.BlockSpec((1, NP, FH), lambda b, t: (b, 0, 0)),
        out_shape=jax.ShapeDtypeStruct((B, NP, FH), jnp.float32),
        scratch_shapes=[pltpu.VMEM((NP, NP), jnp.bfloat16),
                        pltpu.VMEM((NP, NP), jnp.bfloat16),
                        pltpu.VMEM((NP, FH), jnp.float32)],
        compiler_params=pltpu.CompilerParams(
            dimension_semantics=("arbitrary", "arbitrary")),
        interpret=interpret,
    )(*ops)


def _build_s_un(src, dst, ew):
    flat = dst.astype(jnp.int32) * NP + src.astype(jnp.int32)
    return jnp.zeros((NP * NP,), jnp.float32).at[flat].add(ew)


_EPW = 1000
_EPAD = 1008
_HALF = 512
_ZLEN = 32 * NP


def _sc_scatter_body(src_hbm, dst_hbm, ew_hbm, out_hbm,
                     src_v, dst_v, ew_v, idx_v, zbuf, acc, sem):
    c = lax.axis_index("c")
    s = lax.axis_index("s")
    base = s * _EPW
    cp1 = pltpu.async_copy(src_hbm.at[pl.ds(base, _EPW)],
                           src_v.at[pl.ds(0, _EPW)], sem)
    cp2 = pltpu.async_copy(dst_hbm.at[pl.ds(base, _EPW)],
                           dst_v.at[pl.ds(0, _EPW)], sem)
    cp3 = pltpu.async_copy(ew_hbm.at[pl.ds(base, _EPW)],
                           ew_v.at[pl.ds(0, _EPW)], sem)

    def z_body(i, carry):
        zbuf[pl.ds(i * 16, 16)] = jnp.zeros((16,), jnp.float32)
        return carry

    lax.fori_loop(0, _ZLEN // 16, z_body, 0)

    rows_per_sub = _HALF // 16
    pltpu.sync_copy(zbuf, acc.at[pl.ds(s * rows_per_sub * NP, _ZLEN)])

    row0 = c * _HALF
    trash = _HALF * NP
    lane = lax.broadcasted_iota(jnp.int32, (16,), 0)

    @pl.when(s == 15)
    def _():
        pltpu.sync_copy(zbuf.at[pl.ds(0, NP)], acc.at[pl.ds(trash, NP)])

    cp1.wait()
    cp2.wait()
    cp3.wait()

    def idx_body(i, carry):
        d = dst_v[pl.ds(i * 16, 16)]
        sv = src_v[pl.ds(i * 16, 16)]
        dr = d - row0
        ok = (dr >= 0) & (dr < _HALF) & ((i * 16 + lane) < _EPW)
        idx_v[pl.ds(i * 16, 16)] = jnp.where(ok, dr * NP + sv, trash)
        return carry

    lax.fori_loop(0, _EPAD // 16, idx_body, 0)

    plsc.subcore_barrier()
    pltpu.sync_copy(ew_v, acc.at[idx_v], add=True)
    plsc.subcore_barrier()
    pltpu.sync_copy(
        acc.at[pl.ds(s * rows_per_sub * NP, rows_per_sub * NP)],
        out_hbm.at[pl.ds((row0 + s * rows_per_sub) * NP, rows_per_sub * NP)])


def _build_s_un_sc(src, dst, ew):
    mesh = plsc.VectorSubcoreMesh(core_axis_name="c", subcore_axis_name="s")
    f = functools.partial(
        pl.kernel,
        out_type=jax.ShapeDtypeStruct((NP * NP,), jnp.float32),
        mesh=mesh,
        scratch_types=[
            pltpu.VMEM((_EPAD,), jnp.int32),
            pltpu.VMEM((_EPAD,), jnp.int32),
            pltpu.VMEM((_EPAD,), jnp.float32),
            pltpu.VMEM((_EPAD,), jnp.int32),
            pltpu.VMEM((_ZLEN,), jnp.float32),
            pltpu.VMEM_SHARED(((_HALF + 1) * NP,), jnp.float32),
            pltpu.SemaphoreType.DMA,
        ],
    )(_sc_scatter_body)
    return f(src, dst, ew)


def _forward_impl(x, static_edge_weight, W_gsl, b_gsl, W_de1, b_de1, W_de2,
                  b_de2, W_z, b_z, L_z_W, L_z_b, W_r, b_r, L_r_W, L_r_b,
                  W_h, b_h, L_h_W, L_h_b, W_c, b_c, static_edge_index,
                  interpret=False):
    B, N, T = x.shape
    xp = jnp.pad(x, ((0, 0), (0, NP - N), (0, 0)))
    v = xp.reshape(B, NP, NWIN, FW).transpose(0, 2, 1, 3)

    src = static_edge_index[0].astype(jnp.int32)
    dst = static_edge_index[1].astype(jnp.int32)
    s_un = _build_s_un_sc(src, dst, static_edge_weight).reshape(NP, NP)

    s_hi, s_lo = pl.pallas_call(
        _prep_body,
        in_specs=[pl.BlockSpec((NP, NP), lambda: (0, 0))],
        out_specs=[pl.BlockSpec((NP, NP), lambda: (0, 0)),
                   pl.BlockSpec((NP, NP), lambda: (0, 0))],
        out_shape=[jax.ShapeDtypeStruct((NP, NP), jnp.bfloat16),
                   jax.ShapeDtypeStruct((NP, NP), jnp.bfloat16)],
        interpret=interpret,
    )(s_un)

    r1 = lambda a: a.reshape(1, -1)
    weights = (W_gsl[:FW], W_gsl[FW:], r1(b_gsl),
               W_de1, r1(b_de1), W_de2, r1(b_de2),
               W_z, r1(b_z), L_z_W[:FH], L_z_W[FH:], r1(L_z_b),
               W_r, r1(b_r), L_r_W[:FH], L_r_W[FH:], r1(L_r_b),
               W_h, r1(b_h), L_h_W[:FH], L_h_W[FH:], r1(L_h_b))

    h = _run_steps(v, s_hi, s_lo, weights, interpret=interpret)

    hf = h[:, :N, :].reshape(B, N * FH)
    out = pl.pallas_call(
        _cls_body,
        in_specs=[pl.BlockSpec((B, N * FH), lambda: (0, 0)),
                  pl.BlockSpec(W_c.shape, lambda: (0, 0)),
                  pl.BlockSpec((1, b_c.shape[0]), lambda: (0, 0))],
        out_specs=pl.BlockSpec((B, b_c.shape[0]), lambda: (0, 0)),
        out_shape=jax.ShapeDtypeStruct((B, b_c.shape[0]), jnp.float32),
        interpret=interpret,
    )(hf, W_c, r1(b_c))
    return out


def kernel(x, static_edge_weight, W_gsl, b_gsl, W_de1, b_de1, W_de2, b_de2,
           W_z, b_z, L_z_W, L_z_b, W_r, b_r, L_r_W, L_r_b, W_h, b_h, L_h_W,
           L_h_b, W_c, b_c, static_edge_index):
    return _forward_impl(x, static_edge_weight, W_gsl, b_gsl, W_de1, b_de1,
                         W_de2, b_de2, W_z, b_z, L_z_W, L_z_b, W_r, b_r,
                         L_r_W, L_r_b, W_h, b_h, L_h_W, L_h_b, W_c, b_c,
                         static_edge_index)
